# baseline (speedup 1.0000x reference)
import functools

import jax
import jax.numpy as jnp
from jax import lax
from jax.experimental import pallas as pl
from jax.experimental.pallas import tpu as pltpu
from jax.experimental.pallas import tpu_sc as plsc

N = 10000
E = 320000
D = 128
NG = 16
NC = 2
NS = 16
NW = NC * NS
CHUNK = 80
E_PER_W = E // NW
ITERS = E_PER_W // CHUNK
ACC_ROWS = 10240
ROWS_PER_SUB = ACC_ROWS // NS
NSTAGE = ROWS_PER_SUB // CHUNK
BN_SCALE = float(1.0 / (1.0 + 1e-3) ** 0.5)


def _mesh():
  return plsc.VectorSubcoreMesh(core_axis_name="c", subcore_axis_name="s",
                                num_cores=NC, num_subcores=NS)


NBUF = 3
GROUPS = ITERS // NBUF
LOOP_CHUNKS = (GROUPS - 1) * NBUF


def _sc_agg_body(x_hbm, src_hbm, dst_hbm, zrow_hbm, agg_hbm, src_v,
                 dst0, dst1, dst2, rows0, rows1, rows2, agg_sp,
                 sg0, sg1, sg2, sd0, sd1, sd2, ss0, ss1, ss2):
  c = lax.axis_index("c")
  s = lax.axis_index("s")
  w = s * NC + c
  row0 = pl.multiple_of(s * ROWS_PER_SUB, 8)
  out0 = pl.multiple_of(c * ACC_ROWS + row0, 8)
  edge0 = w * E_PER_W
  bufs = [(rows0, sg0, dst0, sd0, ss0), (rows1, sg1, dst1, sd1, ss1),
          (rows2, sg2, dst2, sd2, ss2)]

  pltpu.sync_copy(zrow_hbm, rows0)
  for k in range(NSTAGE):
    pltpu.sync_copy(rows0, agg_sp.at[pl.ds(row0 + k * CHUNK, CHUNK)])
  pltpu.sync_copy(src_hbm.at[pl.ds(edge0, E_PER_W)], src_v)
  plsc.subcore_barrier()

  def _gather(t, rows, sem, dst, dsem, ssem):
    del ssem
    pltpu.async_copy(dst_hbm.at[pl.ds(edge0 + t * CHUNK, CHUNK)], dst, dsem)
    idx = src_v.at[pl.ds(t * CHUNK, CHUNK)]
    pltpu.async_copy(x_hbm.at[idx], rows, sem)

  def _drain(sem, ref):
    pltpu.make_async_copy(x_hbm.at[src_v.at[pl.ds(0, CHUNK)]]
                          if ref.shape == (CHUNK, D) else
                          dst_hbm.at[pl.ds(0, CHUNK)], ref, sem).wait()

  def _scatter_start(rows, sem, dst, dsem, ssem):
    _drain(sem, rows)
    _drain(dsem, dst)
    pltpu.async_copy(rows, agg_sp.at[dst], ssem, add=True)

  def _scatter_drain(rows, sem, dst, dsem, ssem):
    del sem, dsem, dst
    _drain(ssem, rows)

  for i in range(NBUF):
    _gather(i, *bufs[i])

  @pl.loop(0, LOOP_CHUNKS - NBUF + 1, step=NBUF)
  def _edge_loop(t):
    for i in range(NBUF):
      _scatter_start(*bufs[i])
    for i in range(NBUF):
      _scatter_drain(*bufs[i])
      _gather(t + NBUF + i, *bufs[i])

  for ci in range(LOOP_CHUNKS, ITERS):
    _scatter_start(*bufs[ci % NBUF])
    _scatter_drain(*bufs[ci % NBUF])
    if ci + NBUF <= ITERS - 1:
      _gather(ci + NBUF, *bufs[ci % NBUF])

  plsc.subcore_barrier()
  for k in range(NSTAGE):
    pltpu.sync_copy(agg_sp.at[pl.ds(row0 + k * CHUNK, CHUNK)], rows0)
    pltpu.sync_copy(rows0, agg_hbm.at[pl.ds(out0 + k * CHUNK, CHUNK)])


def _sc_aggregate(x, src, dst):
  zrow = jnp.zeros((CHUNK, D), jnp.float32)
  out_type = jax.ShapeDtypeStruct((NC * ACC_ROWS, D), jnp.float32)
  scratch = (
      [pltpu.VMEM((E_PER_W,), jnp.int32)]
      + [pltpu.VMEM((CHUNK,), jnp.int32)] * NBUF
      + [pltpu.VMEM((CHUNK, D), jnp.float32)] * NBUF
      + [pltpu.VMEM_SHARED((ACC_ROWS, D), jnp.float32)]
      + [pltpu.SemaphoreType.DMA] * (3 * NBUF)
  )
  fn = pl.kernel(_sc_agg_body, out_type=out_type, mesh=_mesh(),
                 scratch_types=scratch, name="sc_agg")
  agg = fn(x, src, dst, zrow)
  return agg.reshape(NC, ACC_ROWS, D)


def _sc_degree_body(dst_hbm, zrow_hbm, ones_hbm, deg_hbm,
                    dst_a, dst_b, ones_v, stage_v, deg_sp, sem_da, sem_db,
                    ssem_a, ssem_b):
  c = lax.axis_index("c")
  s = lax.axis_index("s")
  w = s * NC + c
  row0 = pl.multiple_of(s * ROWS_PER_SUB, 8)
  out0 = pl.multiple_of(c * ACC_ROWS + row0, 8)
  edge0 = w * E_PER_W

  pltpu.sync_copy(zrow_hbm, stage_v)
  for k in range(NSTAGE):
    pltpu.sync_copy(stage_v, deg_sp.at[pl.ds(row0 + k * CHUNK, CHUNK)])
  pltpu.sync_copy(ones_hbm, ones_v)
  plsc.subcore_barrier()

  def _load(t, dst, dsem):
    pltpu.async_copy(dst_hbm.at[pl.ds(edge0 + t * CHUNK, CHUNK)], dst, dsem)

  def _sc_start(dst, dsem, ssem):
    pltpu.make_async_copy(dst_hbm.at[pl.ds(0, CHUNK)], dst, dsem).wait()
    pltpu.async_copy(ones_v, deg_sp.at[dst], ssem, add=True)

  def _sc_drain(ssem):
    pltpu.make_async_copy(ones_hbm, ones_v, ssem).wait()

  _load(0, dst_a, sem_da)
  _load(1, dst_b, sem_db)

  @pl.loop(0, ITERS - 4, step=2)
  def _edge_loop(t):
    _sc_start(dst_a, sem_da, ssem_a)
    _sc_start(dst_b, sem_db, ssem_b)
    _sc_drain(ssem_a)
    _load(t + 2, dst_a, sem_da)
    _sc_drain(ssem_b)
    _load(t + 3, dst_b, sem_db)

  _sc_start(dst_a, sem_da, ssem_a)
  _sc_start(dst_b, sem_db, ssem_b)
  _sc_drain(ssem_a)
  _load(ITERS - 1, dst_a, sem_da)
  _sc_drain(ssem_b)
  _sc_start(dst_a, sem_da, ssem_a)
  _sc_drain(ssem_a)

  plsc.subcore_barrier()
  for k in range(NSTAGE):
    pltpu.sync_copy(deg_sp.at[pl.ds(row0 + k * CHUNK, CHUNK)], stage_v)
    pltpu.sync_copy(stage_v, deg_hbm.at[pl.ds(out0 + k * CHUNK, CHUNK)])


def _sc_degree(dst):
  zrow = jnp.zeros((CHUNK, D), jnp.float32)
  ones = jnp.ones((CHUNK, D), jnp.float32)
  out_type = jax.ShapeDtypeStruct((NC * ACC_ROWS, D), jnp.float32)
  scratch = [
      pltpu.VMEM((CHUNK,), jnp.int32),
      pltpu.VMEM((CHUNK,), jnp.int32),
      pltpu.VMEM((CHUNK, D), jnp.float32),
      pltpu.VMEM((CHUNK, D), jnp.float32),
      pltpu.VMEM_SHARED((ACC_ROWS, D), jnp.float32),
      pltpu.SemaphoreType.DMA,
      pltpu.SemaphoreType.DMA,
      pltpu.SemaphoreType.DMA,
      pltpu.SemaphoreType.DMA,
  ]
  fn = pl.kernel(_sc_degree_body, out_type=out_type, mesh=_mesh(),
                 scratch_types=scratch, name="sc_degree")
  deg = fn(dst, zrow, ones)
  return deg.reshape(NC, ACC_ROWS, D)


def _tc_layer_first_body(x_ref, aa_ref, ab_ref, da_ref, db_ref, w_ref,
                         b_ref, g_ref, be_ref, h_ref, r_ref):
  deg = da_ref[0, :, 0:1] + db_ref[0, :, 0:1]
  recip = 1.0 / jnp.maximum(deg, 1.0)
  r_ref[...] = recip
  _tc_layer_core(x_ref, aa_ref, ab_ref, recip, w_ref, b_ref, g_ref, be_ref,
                 h_ref)


def _tc_layer_next_body(x_ref, aa_ref, ab_ref, r_ref, w_ref, b_ref, g_ref,
                        be_ref, h_ref):
  _tc_layer_core(x_ref, aa_ref, ab_ref, r_ref[...], w_ref, b_ref, g_ref,
                 be_ref, h_ref)


def _tc_layer_core(x_ref, aa_ref, ab_ref, recip, w_ref, b_ref, g_ref,
                   be_ref, h_ref):
  agg = (aa_ref[0] + ab_ref[0]) * recip
  hin = jnp.concatenate([x_ref[...], agg], axis=1)
  h = jnp.dot(hin, w_ref[...], preferred_element_type=jnp.float32)
  h = h + b_ref[...]
  ss = jnp.sum(h * h, axis=1, keepdims=True)
  h = h / jnp.maximum(jnp.sqrt(ss), 1e-12)
  h = jnp.maximum(h, 0.0)
  h_ref[...] = h * (g_ref[...] * BN_SCALE) + be_ref[...]


def _tc_layer(x, agg, deg_or_recip, W, b, g, be, first):
  BLK = 1000
  grid = (N // BLK,)
  in_specs = [
      pl.BlockSpec((BLK, D), lambda i: (i, 0)),
      pl.BlockSpec((1, BLK, D), lambda i: (0, i, 0)),
      pl.BlockSpec((1, BLK, D), lambda i: (1, i, 0)),
  ]
  if first:
    in_specs += [
        pl.BlockSpec((1, BLK, D), lambda i: (0, i, 0)),
        pl.BlockSpec((1, BLK, D), lambda i: (1, i, 0)),
    ]
    deg_args = (deg_or_recip, deg_or_recip)
    out_shape = [jax.ShapeDtypeStruct((N, D), jnp.float32),
                 jax.ShapeDtypeStruct((N, 1), jnp.float32)]
    out_specs = [pl.BlockSpec((BLK, D), lambda i: (i, 0)),
                 pl.BlockSpec((BLK, 1), lambda i: (i, 0))]
    body = _tc_layer_first_body
  else:
    in_specs += [pl.BlockSpec((BLK, 1), lambda i: (i, 0))]
    deg_args = (deg_or_recip,)
    out_shape = jax.ShapeDtypeStruct((N, D), jnp.float32)
    out_specs = pl.BlockSpec((BLK, D), lambda i: (i, 0))
    body = _tc_layer_next_body
  in_specs += [
      pl.BlockSpec((2 * D, D), lambda i: (0, 0)),
      pl.BlockSpec((D,), lambda i: (0,)),
      pl.BlockSpec((D,), lambda i: (0,)),
      pl.BlockSpec((D,), lambda i: (0,)),
  ]
  out = pl.pallas_call(
      body, grid=grid, in_specs=in_specs, out_specs=out_specs,
      out_shape=out_shape, name="tc_layer")(
          x, agg, agg, *deg_args, W, b, g, be)
  if first:
    return out[0], out[1]
  return out, None


def _tc_head_body(h_ref, gi_ref, wd1_ref, bd1_ref, wd2_ref, bd2_ref,
                  wo_ref, bo_ref, out_ref):
  gi = gi_ref[...]
  onehot = (gi == lax.broadcasted_iota(jnp.int32, (N, NG), 1))
  onehot = onehot.astype(jnp.float32)
  pooled = lax.dot_general(onehot, h_ref[...], (((0,), (0,)), ((), ())),
                           preferred_element_type=jnp.float32)
  counts = lax.dot_general(onehot, jnp.ones((N, 1), jnp.float32),
                           (((0,), (0,)), ((), ())),
                           preferred_element_type=jnp.float32)
  pooled = pooled / jnp.maximum(counts, 1.0)
  z = jnp.dot(pooled, wd1_ref[...], preferred_element_type=jnp.float32)
  z = jnp.maximum(z + bd1_ref[...], 0.0)
  z = jnp.dot(z, wd2_ref[...], preferred_element_type=jnp.float32)
  z = jnp.maximum(z + bd2_ref[...], 0.0)
  logits = jnp.dot(z, wo_ref[...], preferred_element_type=jnp.float32)
  logits = logits + bo_ref[...]
  m = jnp.max(logits, axis=1, keepdims=True)
  e = jnp.exp(logits - m)
  out_ref[...] = e / jnp.sum(e, axis=1, keepdims=True)


def _tc_head(h3, gi, Wd1, bd1, Wd2, bd2, Wo, bo):
  return pl.pallas_call(
      _tc_head_body,
      out_shape=jax.ShapeDtypeStruct((NG, NG), jnp.float32),
      name="tc_head")(h3, gi, Wd1, bd1, Wd2, bd2, Wo, bo)


def kernel(x, edge_index, graph_idx, W1, b1, g1, be1, W2, b2, g2, be2,
           W3, b3, g3, be3, Wd1, bd1, Wd2, bd2, Wo, bo):
  src = edge_index[0].astype(jnp.int32)
  dst = edge_index[1].astype(jnp.int32)
  gi = graph_idx.astype(jnp.int32).reshape(N, 1)

  deg = _sc_degree(dst)
  agg1 = _sc_aggregate(x, src, dst)
  h1, recip = _tc_layer(x, agg1, deg, W1, b1, g1, be1, first=True)
  agg2 = _sc_aggregate(h1, src, dst)
  h2, _ = _tc_layer(h1, agg2, recip, W2, b2, g2, be2, first=False)
  agg3 = _sc_aggregate(h2, src, dst)
  h3, _ = _tc_layer(h2, agg3, recip, W3, b3, g3, be3, first=False)
  return _tc_head(h3, gi, Wd1, bd1, Wd2, bd2, Wo, bo)


# speedup vs baseline: 1.2019x; 1.2019x over previous
import functools

import jax
import jax.numpy as jnp
from jax import lax
from jax.experimental import pallas as pl
from jax.experimental.pallas import tpu as pltpu
from jax.experimental.pallas import tpu_sc as plsc

N = 10000
E = 320000
D = 128
NG = 16
NC = 2
NS = 16
NW = NC * NS
CHUNK = 80
E_PER_W = E // NW
ITERS = E_PER_W // CHUNK
ACC_ROWS = 10240
ROWS_PER_SUB = ACC_ROWS // NS
NSTAGE = ROWS_PER_SUB // CHUNK
BN_SCALE = float(1.0 / (1.0 + 1e-3) ** 0.5)


def _mesh():
  return plsc.VectorSubcoreMesh(core_axis_name="c", subcore_axis_name="s",
                                num_cores=NC, num_subcores=NS)


NBUF = 3
GROUPS = ITERS // NBUF
LOOP_CHUNKS = (GROUPS - 1) * NBUF


def _sc_agg_body(x_hbm, src_hbm, dst_hbm, zrow_hbm, agg_hbm, src_v,
                 dst0, dst1, dst2, rows0, rows1, rows2, agg_sp,
                 sg0, sg1, sg2, sd0, sd1, sd2):
  c = lax.axis_index("c")
  s = lax.axis_index("s")
  w = s * NC + c
  row0 = pl.multiple_of(s * ROWS_PER_SUB, 8)
  out0 = pl.multiple_of(c * ACC_ROWS + row0, 8)
  edge0 = w * E_PER_W
  bufs = [(rows0, sg0, dst0, sd0), (rows1, sg1, dst1, sd1),
          (rows2, sg2, dst2, sd2)]

  pltpu.sync_copy(zrow_hbm, rows0)
  for k in range(NSTAGE):
    pltpu.sync_copy(rows0, agg_sp.at[pl.ds(row0 + k * CHUNK, CHUNK)])
  pltpu.sync_copy(src_hbm.at[pl.ds(edge0, E_PER_W)], src_v)
  plsc.subcore_barrier()

  def _gather(t, rows, sem, dst, dsem):
    pltpu.async_copy(dst_hbm.at[pl.ds(edge0 + t * CHUNK, CHUNK)], dst, dsem)
    idx = src_v.at[pl.ds(t * CHUNK, CHUNK)]
    pltpu.async_copy(x_hbm.at[idx], rows, sem)

  def _scatter(t, rows, sem, dst, dsem):
    pltpu.make_async_copy(x_hbm.at[src_v.at[pl.ds(0, CHUNK)]], rows,
                          sem).wait()
    pltpu.make_async_copy(dst_hbm.at[pl.ds(0, CHUNK)], dst, dsem).wait()
    pltpu.sync_copy(rows, agg_sp.at[dst], add=True)

  for i in range(NBUF):
    _gather(i, *bufs[i])

  @pl.loop(0, LOOP_CHUNKS - NBUF + 1, step=NBUF)
  def _edge_loop(t):
    for i in range(NBUF):
      _scatter(t + i, *bufs[i])
      _gather(t + NBUF + i, *bufs[i])

  for ci in range(LOOP_CHUNKS, ITERS):
    _scatter(ci, *bufs[ci % NBUF])
    if ci + NBUF <= ITERS - 1:
      _gather(ci + NBUF, *bufs[ci % NBUF])

  plsc.subcore_barrier()
  for k in range(NSTAGE):
    pltpu.sync_copy(agg_sp.at[pl.ds(row0 + k * CHUNK, CHUNK)], rows0)
    pltpu.sync_copy(rows0, agg_hbm.at[pl.ds(out0 + k * CHUNK, CHUNK)])


def _sc_aggregate(x, src, dst):
  zrow = jnp.zeros((CHUNK, D), jnp.float32)
  out_type = jax.ShapeDtypeStruct((NC * ACC_ROWS, D), jnp.float32)
  scratch = (
      [pltpu.VMEM((E_PER_W,), jnp.int32)]
      + [pltpu.VMEM((CHUNK,), jnp.int32)] * NBUF
      + [pltpu.VMEM((CHUNK, D), jnp.float32)] * NBUF
      + [pltpu.VMEM_SHARED((ACC_ROWS, D), jnp.float32)]
      + [pltpu.SemaphoreType.DMA] * (2 * NBUF)
  )
  fn = pl.kernel(_sc_agg_body, out_type=out_type, mesh=_mesh(),
                 scratch_types=scratch, name="sc_agg")
  agg = fn(x, src, dst, zrow)
  return agg.reshape(NC, ACC_ROWS, D)


def _sc_degree_body(dst_hbm, zrow_hbm, ones_hbm, deg_hbm,
                    dst_a, dst_b, ones_v, stage_v, deg_sp, sem_da, sem_db):
  c = lax.axis_index("c")
  s = lax.axis_index("s")
  w = s * NC + c
  row0 = pl.multiple_of(s * ROWS_PER_SUB, 8)
  out0 = pl.multiple_of(c * ACC_ROWS + row0, 8)
  edge0 = w * E_PER_W

  pltpu.sync_copy(zrow_hbm, stage_v)
  for k in range(NSTAGE):
    pltpu.sync_copy(stage_v, deg_sp.at[pl.ds(row0 + k * CHUNK, CHUNK)])
  pltpu.sync_copy(ones_hbm, ones_v)
  plsc.subcore_barrier()

  def _load(t, dst, dsem):
    pltpu.async_copy(dst_hbm.at[pl.ds(edge0 + t * CHUNK, CHUNK)], dst, dsem)

  def _scatter(dst, dsem):
    pltpu.make_async_copy(dst_hbm.at[pl.ds(0, CHUNK)], dst, dsem).wait()
    pltpu.sync_copy(ones_v, deg_sp.at[dst], add=True)

  _load(0, dst_a, sem_da)

  @pl.loop(0, ITERS - 1, step=2)
  def _edge_loop(t):
    _load(t + 1, dst_b, sem_db)
    _scatter(dst_a, sem_da)
    _load(t + 2, dst_a, sem_da)
    _scatter(dst_b, sem_db)

  _scatter(dst_a, sem_da)

  plsc.subcore_barrier()
  for k in range(NSTAGE):
    pltpu.sync_copy(deg_sp.at[pl.ds(row0 + k * CHUNK, CHUNK)], stage_v)
    pltpu.sync_copy(stage_v, deg_hbm.at[pl.ds(out0 + k * CHUNK, CHUNK)])


def _sc_degree(dst):
  zrow = jnp.zeros((CHUNK, D), jnp.float32)
  ones = jnp.ones((CHUNK, D), jnp.float32)
  out_type = jax.ShapeDtypeStruct((NC * ACC_ROWS, D), jnp.float32)
  scratch = [
      pltpu.VMEM((CHUNK,), jnp.int32),
      pltpu.VMEM((CHUNK,), jnp.int32),
      pltpu.VMEM((CHUNK, D), jnp.float32),
      pltpu.VMEM((CHUNK, D), jnp.float32),
      pltpu.VMEM_SHARED((ACC_ROWS, D), jnp.float32),
      pltpu.SemaphoreType.DMA,
      pltpu.SemaphoreType.DMA,
  ]
  fn = pl.kernel(_sc_degree_body, out_type=out_type, mesh=_mesh(),
                 scratch_types=scratch, name="sc_degree")
  deg = fn(dst, zrow, ones)
  return deg.reshape(NC, ACC_ROWS, D)


def _tc_layer_first_body(x_ref, aa_ref, ab_ref, da_ref, db_ref, w_ref,
                         b_ref, g_ref, be_ref, h_ref, r_ref):
  deg = da_ref[0, :, 0:1] + db_ref[0, :, 0:1]
  recip = 1.0 / jnp.maximum(deg, 1.0)
  r_ref[...] = recip
  _tc_layer_core(x_ref, aa_ref, ab_ref, recip, w_ref, b_ref, g_ref, be_ref,
                 h_ref)


def _tc_layer_next_body(x_ref, aa_ref, ab_ref, r_ref, w_ref, b_ref, g_ref,
                        be_ref, h_ref):
  _tc_layer_core(x_ref, aa_ref, ab_ref, r_ref[...], w_ref, b_ref, g_ref,
                 be_ref, h_ref)


def _tc_layer_core(x_ref, aa_ref, ab_ref, recip, w_ref, b_ref, g_ref,
                   be_ref, h_ref):
  agg = (aa_ref[0] + ab_ref[0]) * recip
  hin = jnp.concatenate([x_ref[...], agg], axis=1)
  h = jnp.dot(hin, w_ref[...], preferred_element_type=jnp.float32)
  h = h + b_ref[...]
  ss = jnp.sum(h * h, axis=1, keepdims=True)
  h = h / jnp.maximum(jnp.sqrt(ss), 1e-12)
  h = jnp.maximum(h, 0.0)
  h_ref[...] = h * (g_ref[...] * BN_SCALE) + be_ref[...]


def _tc_layer(x, agg, deg_or_recip, W, b, g, be, first):
  BLK = 1000
  grid = (N // BLK,)
  in_specs = [
      pl.BlockSpec((BLK, D), lambda i: (i, 0)),
      pl.BlockSpec((1, BLK, D), lambda i: (0, i, 0)),
      pl.BlockSpec((1, BLK, D), lambda i: (1, i, 0)),
  ]
  if first:
    in_specs += [
        pl.BlockSpec((1, BLK, D), lambda i: (0, i, 0)),
        pl.BlockSpec((1, BLK, D), lambda i: (1, i, 0)),
    ]
    deg_args = (deg_or_recip, deg_or_recip)
    out_shape = [jax.ShapeDtypeStruct((N, D), jnp.float32),
                 jax.ShapeDtypeStruct((N, 1), jnp.float32)]
    out_specs = [pl.BlockSpec((BLK, D), lambda i: (i, 0)),
                 pl.BlockSpec((BLK, 1), lambda i: (i, 0))]
    body = _tc_layer_first_body
  else:
    in_specs += [pl.BlockSpec((BLK, 1), lambda i: (i, 0))]
    deg_args = (deg_or_recip,)
    out_shape = jax.ShapeDtypeStruct((N, D), jnp.float32)
    out_specs = pl.BlockSpec((BLK, D), lambda i: (i, 0))
    body = _tc_layer_next_body
  in_specs += [
      pl.BlockSpec((2 * D, D), lambda i: (0, 0)),
      pl.BlockSpec((D,), lambda i: (0,)),
      pl.BlockSpec((D,), lambda i: (0,)),
      pl.BlockSpec((D,), lambda i: (0,)),
  ]
  out = pl.pallas_call(
      body, grid=grid, in_specs=in_specs, out_specs=out_specs,
      out_shape=out_shape, name="tc_layer")(
          x, agg, agg, *deg_args, W, b, g, be)
  if first:
    return out[0], out[1]
  return out, None


def _tc_head_body(h_ref, gi_ref, wd1_ref, bd1_ref, wd2_ref, bd2_ref,
                  wo_ref, bo_ref, out_ref):
  gi = gi_ref[...]
  onehot = (gi == lax.broadcasted_iota(jnp.int32, (N, NG), 1))
  onehot = onehot.astype(jnp.float32)
  pooled = lax.dot_general(onehot, h_ref[...], (((0,), (0,)), ((), ())),
                           preferred_element_type=jnp.float32)
  counts = lax.dot_general(onehot, jnp.ones((N, 1), jnp.float32),
                           (((0,), (0,)), ((), ())),
                           preferred_element_type=jnp.float32)
  pooled = pooled / jnp.maximum(counts, 1.0)
  z = jnp.dot(pooled, wd1_ref[...], preferred_element_type=jnp.float32)
  z = jnp.maximum(z + bd1_ref[...], 0.0)
  z = jnp.dot(z, wd2_ref[...], preferred_element_type=jnp.float32)
  z = jnp.maximum(z + bd2_ref[...], 0.0)
  logits = jnp.dot(z, wo_ref[...], preferred_element_type=jnp.float32)
  logits = logits + bo_ref[...]
  m = jnp.max(logits, axis=1, keepdims=True)
  e = jnp.exp(logits - m)
  out_ref[...] = e / jnp.sum(e, axis=1, keepdims=True)


def _tc_head(h3, gi, Wd1, bd1, Wd2, bd2, Wo, bo):
  return pl.pallas_call(
      _tc_head_body,
      out_shape=jax.ShapeDtypeStruct((NG, NG), jnp.float32),
      name="tc_head")(h3, gi, Wd1, bd1, Wd2, bd2, Wo, bo)


def kernel(x, edge_index, graph_idx, W1, b1, g1, be1, W2, b2, g2, be2,
           W3, b3, g3, be3, Wd1, bd1, Wd2, bd2, Wo, bo):
  src = edge_index[0].astype(jnp.int32)
  dst = edge_index[1].astype(jnp.int32)
  gi = graph_idx.astype(jnp.int32).reshape(N, 1)

  deg = _sc_degree(dst)
  agg1 = _sc_aggregate(x, src, dst)
  h1, recip = _tc_layer(x, agg1, deg, W1, b1, g1, be1, first=True)
  agg2 = _sc_aggregate(h1, src, dst)
  h2, _ = _tc_layer(h1, agg2, recip, W2, b2, g2, be2, first=False)
  agg3 = _sc_aggregate(h2, src, dst)
  h3, _ = _tc_layer(h2, agg3, recip, W3, b3, g3, be3, first=False)
  return _tc_head(h3, gi, Wd1, bd1, Wd2, bd2, Wo, bo)
